# Initial kernel scaffold; baseline (speedup 1.0000x reference)
import jax
import jax.numpy as jnp
from jax.experimental import pallas as pl


def kernel(x, edge_index, W, b):
    raise NotImplementedError("write your pallas kernel here")


# speedup vs baseline: 15.7254x; 15.7254x over previous
import functools

import jax
import jax.numpy as jnp
from jax import lax
from jax.experimental import pallas as pl
from jax.experimental.pallas import tpu as pltpu
from jax.experimental.pallas import tpu_sc as plsc

N = 10000
IN_CH = 128
OUT_CH = 32

NC, NS = 2, 16
NW = NC * NS
N_PAD = 10240
ROWS_PER_TILE = N_PAD // NS
CHUNK = 128
E = 320000
EPW = 10240
E_PAD = EPW * NW
N_CHUNKS = EPW // CHUNK

_mesh = plsc.VectorSubcoreMesh(core_axis_name="c", subcore_axis_name="s")
_sc_params = pltpu.CompilerParams(use_tc_tiling_on_sc=False)


@functools.partial(
    pl.kernel,
    out_type=jax.ShapeDtypeStruct((NC * N_PAD,), jnp.float32),
    mesh=_mesh,
    scratch_types=[
        pltpu.VMEM((CHUNK,), jnp.int32),
        pltpu.VMEM((CHUNK,), jnp.float32),
        pltpu.VMEM_SHARED((N_PAD,), jnp.float32),
    ],
    compiler_params=_sc_params,
)
def _deg_kernel(dst_hbm, ones_hbm, zeros_hbm, out_hbm, dst_v, ones_v, acc_sh):
    c = lax.axis_index("c")
    s = lax.axis_index("s")
    wid = s * NC + c
    row0 = s * ROWS_PER_TILE
    pltpu.sync_copy(zeros_hbm, acc_sh.at[pl.ds(row0, ROWS_PER_TILE)])
    pltpu.sync_copy(ones_hbm, ones_v)
    plsc.subcore_barrier()
    base = wid * EPW

    def body(j, carry):
        pltpu.sync_copy(dst_hbm.at[pl.ds(base + j * CHUNK, CHUNK)], dst_v)
        pltpu.sync_copy(ones_v, acc_sh.at[dst_v], add=True)
        return carry

    lax.fori_loop(0, N_CHUNKS, body, 0)
    plsc.subcore_barrier()
    pltpu.sync_copy(
        acc_sh.at[pl.ds(row0, ROWS_PER_TILE)],
        out_hbm.at[pl.ds(c * N_PAD + row0, ROWS_PER_TILE)],
    )


@functools.partial(
    pl.kernel,
    out_type=jax.ShapeDtypeStruct((NC * N_PAD, OUT_CH), jnp.float32),
    mesh=_mesh,
    scratch_types=[
        pltpu.VMEM((CHUNK,), jnp.int32),
        pltpu.VMEM((CHUNK,), jnp.int32),
        pltpu.VMEM((CHUNK, OUT_CH), jnp.float32),
        pltpu.VMEM_SHARED((N_PAD, OUT_CH), jnp.float32),
        pltpu.SemaphoreType.DMA,
    ],
    compiler_params=_sc_params,
)
def _hop_kernel(y_hbm, src_hbm, dst_hbm, zeros_hbm, out_hbm,
                src_v, dst_v, rows_v, acc_sh, sem):
    c = lax.axis_index("c")
    s = lax.axis_index("s")
    wid = s * NC + c
    row0 = s * ROWS_PER_TILE
    pltpu.sync_copy(zeros_hbm, acc_sh.at[pl.ds(row0, ROWS_PER_TILE)])
    plsc.subcore_barrier()
    base = wid * EPW

    def body(j, carry):
        off = base + j * CHUNK
        pltpu.sync_copy(src_hbm.at[pl.ds(off, CHUNK)], src_v)
        pltpu.sync_copy(dst_hbm.at[pl.ds(off, CHUNK)], dst_v)
        pltpu.async_copy(y_hbm.at[src_v], rows_v, sem).wait()
        pltpu.sync_copy(rows_v, acc_sh.at[dst_v], add=True)
        return carry

    lax.fori_loop(0, N_CHUNKS, body, 0)
    plsc.subcore_barrier()
    pltpu.sync_copy(
        acc_sh.at[pl.ds(row0, ROWS_PER_TILE)],
        out_hbm.at[pl.ds(c * N_PAD + row0, ROWS_PER_TILE)],
    )


_ROWS = 2000
_GRID = N // _ROWS


def _mm_body(x_ref, w_ref, o_ref):
    o_ref[...] = jnp.dot(x_ref[...], w_ref[...],
                         preferred_element_type=jnp.float32)


_matmul = pl.pallas_call(
    _mm_body,
    grid=(_GRID,),
    in_specs=[
        pl.BlockSpec((_ROWS, IN_CH), lambda i: (i, 0)),
        pl.BlockSpec((IN_CH, OUT_CH), lambda i: (0, 0)),
    ],
    out_specs=pl.BlockSpec((_ROWS, OUT_CH), lambda i: (i, 0)),
    out_shape=jax.ShapeDtypeStruct((N, OUT_CH), jnp.float32),
)


def _scale0_body(h_ref, c0_ref, c1_ref, y_ref, deg_ref):
    deg = c0_ref[...] + c1_ref[...] + 1.0
    deg_ref[...] = deg
    y_ref[...] = h_ref[...] * lax.rsqrt(deg)


_scale0 = pl.pallas_call(
    _scale0_body,
    grid=(_GRID,),
    in_specs=[
        pl.BlockSpec((_ROWS, OUT_CH), lambda i: (i, 0)),
        pl.BlockSpec((_ROWS, 1), lambda i: (i, 0)),
        pl.BlockSpec((_ROWS, 1), lambda i: (i, 0)),
    ],
    out_specs=[
        pl.BlockSpec((_ROWS, OUT_CH), lambda i: (i, 0)),
        pl.BlockSpec((_ROWS, 1), lambda i: (i, 0)),
    ],
    out_shape=[
        jax.ShapeDtypeStruct((N, OUT_CH), jnp.float32),
        jax.ShapeDtypeStruct((N, 1), jnp.float32),
    ],
)


def _mid_body(p0_ref, p1_ref, y_ref, deg_ref, o_ref):
    s = p0_ref[...] + p1_ref[...] + y_ref[...]
    o_ref[...] = s / deg_ref[...]


_mid = pl.pallas_call(
    _mid_body,
    grid=(_GRID,),
    in_specs=[
        pl.BlockSpec((_ROWS, OUT_CH), lambda i: (i, 0)),
        pl.BlockSpec((_ROWS, OUT_CH), lambda i: (i, 0)),
        pl.BlockSpec((_ROWS, OUT_CH), lambda i: (i, 0)),
        pl.BlockSpec((_ROWS, 1), lambda i: (i, 0)),
    ],
    out_specs=pl.BlockSpec((_ROWS, OUT_CH), lambda i: (i, 0)),
    out_shape=jax.ShapeDtypeStruct((N, OUT_CH), jnp.float32),
)


def _final_body(p0_ref, p1_ref, y_ref, deg_ref, b_ref, o_ref):
    s = p0_ref[...] + p1_ref[...] + y_ref[...]
    logits = s * lax.rsqrt(deg_ref[...]) + b_ref[...]
    m = jnp.max(logits, axis=1, keepdims=True)
    z = jnp.exp(logits - m)
    lse = jnp.log(jnp.sum(z, axis=1, keepdims=True)) + m
    o_ref[...] = logits - lse


_final = pl.pallas_call(
    _final_body,
    grid=(_GRID,),
    in_specs=[
        pl.BlockSpec((_ROWS, OUT_CH), lambda i: (i, 0)),
        pl.BlockSpec((_ROWS, OUT_CH), lambda i: (i, 0)),
        pl.BlockSpec((_ROWS, OUT_CH), lambda i: (i, 0)),
        pl.BlockSpec((_ROWS, 1), lambda i: (i, 0)),
        pl.BlockSpec((1, OUT_CH), lambda i: (0, 0)),
    ],
    out_specs=pl.BlockSpec((_ROWS, OUT_CH), lambda i: (i, 0)),
    out_shape=jax.ShapeDtypeStruct((N, OUT_CH), jnp.float32),
)


@jax.jit
def kernel(x, edge_index, W, b):
    src = edge_index[0].astype(jnp.int32)
    dst = edge_index[1].astype(jnp.int32)
    pad = E_PAD - E
    src_p = jnp.concatenate([src, jnp.zeros((pad,), jnp.int32)])
    dst_p = jnp.concatenate([dst, jnp.full((pad,), N, jnp.int32)])

    ones_c = jnp.ones((CHUNK,), jnp.float32)
    zeros_1 = jnp.zeros((ROWS_PER_TILE,), jnp.float32)
    zeros_2 = jnp.zeros((ROWS_PER_TILE, OUT_CH), jnp.float32)

    h0 = _matmul(x, W)

    cnt = _deg_kernel(dst_p, ones_c, zeros_1)
    cnt = cnt.reshape(NC, N_PAD)
    c0 = cnt[0, :N, None]
    c1 = cnt[1, :N, None]

    y0, deg = _scale0(h0, c0, c1)

    p1 = _hop_kernel(y0, src_p, dst_p, zeros_2).reshape(NC, N_PAD, OUT_CH)
    y1 = _mid(p1[0, :N], p1[1, :N], y0, deg)

    p2 = _hop_kernel(y1, src_p, dst_p, zeros_2).reshape(NC, N_PAD, OUT_CH)
    out = _final(p2[0, :N], p2[1, :N], y1, deg, b.reshape(1, OUT_CH))
    return out


# speedup vs baseline: 24.2071x; 1.5394x over previous
import functools

import jax
import jax.numpy as jnp
from jax import lax
from jax.experimental import pallas as pl
from jax.experimental.pallas import tpu as pltpu
from jax.experimental.pallas import tpu_sc as plsc

N = 10000
IN_CH = 128
OUT_CH = 32

NC, NS = 2, 16
NW = NC * NS
N_PAD = 10240
ROWS_PER_TILE = N_PAD // NS
CHUNK = 1024
E = 320000
EPW = 10240
E_PAD = EPW * NW
N_CHUNKS = EPW // CHUNK

_mesh = plsc.VectorSubcoreMesh(core_axis_name="c", subcore_axis_name="s")
_sc_params = pltpu.CompilerParams(use_tc_tiling_on_sc=False)


@functools.partial(
    pl.kernel,
    out_type=jax.ShapeDtypeStruct((NC * N_PAD,), jnp.float32),
    mesh=_mesh,
    scratch_types=[
        pltpu.VMEM((CHUNK,), jnp.int32),
        pltpu.VMEM((CHUNK,), jnp.float32),
        pltpu.VMEM_SHARED((N_PAD,), jnp.float32),
    ],
    compiler_params=_sc_params,
)
def _deg_kernel(dst_hbm, ones_hbm, zeros_hbm, out_hbm, dst_v, ones_v, acc_sh):
    c = lax.axis_index("c")
    s = lax.axis_index("s")
    wid = s * NC + c
    row0 = s * ROWS_PER_TILE
    pltpu.sync_copy(zeros_hbm, acc_sh.at[pl.ds(row0, ROWS_PER_TILE)])
    pltpu.sync_copy(ones_hbm, ones_v)
    plsc.subcore_barrier()
    base = wid * EPW

    def body(j, carry):
        pltpu.sync_copy(dst_hbm.at[pl.ds(base + j * CHUNK, CHUNK)], dst_v)
        pltpu.sync_copy(ones_v, acc_sh.at[dst_v], add=True)
        return carry

    lax.fori_loop(0, N_CHUNKS, body, 0)
    plsc.subcore_barrier()
    pltpu.sync_copy(
        acc_sh.at[pl.ds(row0, ROWS_PER_TILE)],
        out_hbm.at[pl.ds(c * N_PAD + row0, ROWS_PER_TILE)],
    )


@functools.partial(
    pl.kernel,
    out_type=jax.ShapeDtypeStruct((NC * N_PAD, OUT_CH), jnp.float32),
    mesh=_mesh,
    scratch_types=[
        pltpu.VMEM((CHUNK,), jnp.int32),
        pltpu.VMEM((CHUNK,), jnp.int32),
        pltpu.VMEM((CHUNK, OUT_CH), jnp.float32),
        pltpu.VMEM_SHARED((N_PAD, OUT_CH), jnp.float32),
        pltpu.SemaphoreType.DMA,
    ],
    compiler_params=_sc_params,
)
def _hop_kernel(y_hbm, src_hbm, dst_hbm, zeros_hbm, out_hbm,
                src_v, dst_v, rows_v, acc_sh, sem):
    c = lax.axis_index("c")
    s = lax.axis_index("s")
    wid = s * NC + c
    row0 = s * ROWS_PER_TILE
    pltpu.sync_copy(zeros_hbm, acc_sh.at[pl.ds(row0, ROWS_PER_TILE)])
    plsc.subcore_barrier()
    base = wid * EPW

    def body(j, carry):
        off = base + j * CHUNK
        pltpu.sync_copy(src_hbm.at[pl.ds(off, CHUNK)], src_v)
        pltpu.sync_copy(dst_hbm.at[pl.ds(off, CHUNK)], dst_v)
        pltpu.async_copy(y_hbm.at[src_v], rows_v, sem).wait()
        pltpu.sync_copy(rows_v, acc_sh.at[dst_v], add=True)
        return carry

    lax.fori_loop(0, N_CHUNKS, body, 0)
    plsc.subcore_barrier()
    pltpu.sync_copy(
        acc_sh.at[pl.ds(row0, ROWS_PER_TILE)],
        out_hbm.at[pl.ds(c * N_PAD + row0, ROWS_PER_TILE)],
    )


_ROWS = 2000
_GRID = N // _ROWS


def _mm_body(x_ref, w_ref, o_ref):
    o_ref[...] = jnp.dot(x_ref[...], w_ref[...],
                         preferred_element_type=jnp.float32)


_matmul = pl.pallas_call(
    _mm_body,
    grid=(_GRID,),
    in_specs=[
        pl.BlockSpec((_ROWS, IN_CH), lambda i: (i, 0)),
        pl.BlockSpec((IN_CH, OUT_CH), lambda i: (0, 0)),
    ],
    out_specs=pl.BlockSpec((_ROWS, OUT_CH), lambda i: (i, 0)),
    out_shape=jax.ShapeDtypeStruct((N, OUT_CH), jnp.float32),
)


def _scale0_body(h_ref, c0_ref, c1_ref, y_ref, deg_ref):
    deg = c0_ref[...] + c1_ref[...] + 1.0
    deg_ref[...] = deg
    y_ref[...] = h_ref[...] * lax.rsqrt(deg)


_scale0 = pl.pallas_call(
    _scale0_body,
    grid=(_GRID,),
    in_specs=[
        pl.BlockSpec((_ROWS, OUT_CH), lambda i: (i, 0)),
        pl.BlockSpec((_ROWS, 1), lambda i: (i, 0)),
        pl.BlockSpec((_ROWS, 1), lambda i: (i, 0)),
    ],
    out_specs=[
        pl.BlockSpec((_ROWS, OUT_CH), lambda i: (i, 0)),
        pl.BlockSpec((_ROWS, 1), lambda i: (i, 0)),
    ],
    out_shape=[
        jax.ShapeDtypeStruct((N, OUT_CH), jnp.float32),
        jax.ShapeDtypeStruct((N, 1), jnp.float32),
    ],
)


def _mid_body(p0_ref, p1_ref, y_ref, deg_ref, o_ref):
    s = p0_ref[...] + p1_ref[...] + y_ref[...]
    o_ref[...] = s / deg_ref[...]


_mid = pl.pallas_call(
    _mid_body,
    grid=(_GRID,),
    in_specs=[
        pl.BlockSpec((_ROWS, OUT_CH), lambda i: (i, 0)),
        pl.BlockSpec((_ROWS, OUT_CH), lambda i: (i, 0)),
        pl.BlockSpec((_ROWS, OUT_CH), lambda i: (i, 0)),
        pl.BlockSpec((_ROWS, 1), lambda i: (i, 0)),
    ],
    out_specs=pl.BlockSpec((_ROWS, OUT_CH), lambda i: (i, 0)),
    out_shape=jax.ShapeDtypeStruct((N, OUT_CH), jnp.float32),
)


def _final_body(p0_ref, p1_ref, y_ref, deg_ref, b_ref, o_ref):
    s = p0_ref[...] + p1_ref[...] + y_ref[...]
    logits = s * lax.rsqrt(deg_ref[...]) + b_ref[...]
    m = jnp.max(logits, axis=1, keepdims=True)
    z = jnp.exp(logits - m)
    lse = jnp.log(jnp.sum(z, axis=1, keepdims=True)) + m
    o_ref[...] = logits - lse


_final = pl.pallas_call(
    _final_body,
    grid=(_GRID,),
    in_specs=[
        pl.BlockSpec((_ROWS, OUT_CH), lambda i: (i, 0)),
        pl.BlockSpec((_ROWS, OUT_CH), lambda i: (i, 0)),
        pl.BlockSpec((_ROWS, OUT_CH), lambda i: (i, 0)),
        pl.BlockSpec((_ROWS, 1), lambda i: (i, 0)),
        pl.BlockSpec((1, OUT_CH), lambda i: (0, 0)),
    ],
    out_specs=pl.BlockSpec((_ROWS, OUT_CH), lambda i: (i, 0)),
    out_shape=jax.ShapeDtypeStruct((N, OUT_CH), jnp.float32),
)


@jax.jit
def kernel(x, edge_index, W, b):
    src = edge_index[0].astype(jnp.int32)
    dst = edge_index[1].astype(jnp.int32)
    pad = E_PAD - E
    src_p = jnp.concatenate([src, jnp.zeros((pad,), jnp.int32)])
    dst_p = jnp.concatenate([dst, jnp.full((pad,), N, jnp.int32)])

    ones_c = jnp.ones((CHUNK,), jnp.float32)
    zeros_1 = jnp.zeros((ROWS_PER_TILE,), jnp.float32)
    zeros_2 = jnp.zeros((ROWS_PER_TILE, OUT_CH), jnp.float32)

    h0 = _matmul(x, W)

    cnt = _deg_kernel(dst_p, ones_c, zeros_1)
    cnt = cnt.reshape(NC, N_PAD)
    c0 = cnt[0, :N, None]
    c1 = cnt[1, :N, None]

    y0, deg = _scale0(h0, c0, c1)

    p1 = _hop_kernel(y0, src_p, dst_p, zeros_2).reshape(NC, N_PAD, OUT_CH)
    y1 = _mid(p1[0, :N], p1[1, :N], y0, deg)

    p2 = _hop_kernel(y1, src_p, dst_p, zeros_2).reshape(NC, N_PAD, OUT_CH)
    out = _final(p2[0, :N], p2[1, :N], y1, deg, b.reshape(1, OUT_CH))
    return out


# speedup vs baseline: 24.9625x; 1.0312x over previous
import functools

import jax
import jax.numpy as jnp
from jax import lax
from jax.experimental import pallas as pl
from jax.experimental.pallas import tpu as pltpu
from jax.experimental.pallas import tpu_sc as plsc

N = 10000
IN_CH = 128
OUT_CH = 32

NC, NS = 2, 16
NW = NC * NS
N_PAD = 10240
ROWS_PER_TILE = N_PAD // NS
CHUNK = 2048
E = 320000
EPW = 10240
E_PAD = EPW * NW
N_CHUNKS = EPW // CHUNK

_mesh = plsc.VectorSubcoreMesh(core_axis_name="c", subcore_axis_name="s")
_sc_params = pltpu.CompilerParams(use_tc_tiling_on_sc=False)


@functools.partial(
    pl.kernel,
    out_type=jax.ShapeDtypeStruct((NC * N_PAD,), jnp.float32),
    mesh=_mesh,
    scratch_types=[
        pltpu.VMEM((CHUNK,), jnp.int32),
        pltpu.VMEM((CHUNK,), jnp.float32),
        pltpu.VMEM_SHARED((N_PAD,), jnp.float32),
    ],
    compiler_params=_sc_params,
)
def _deg_kernel(dst_hbm, ones_hbm, zeros_hbm, out_hbm, dst_v, ones_v, acc_sh):
    c = lax.axis_index("c")
    s = lax.axis_index("s")
    wid = s * NC + c
    row0 = s * ROWS_PER_TILE
    pltpu.sync_copy(zeros_hbm, acc_sh.at[pl.ds(row0, ROWS_PER_TILE)])
    pltpu.sync_copy(ones_hbm, ones_v)
    plsc.subcore_barrier()
    base = wid * EPW

    def body(j, carry):
        pltpu.sync_copy(dst_hbm.at[pl.ds(base + j * CHUNK, CHUNK)], dst_v)
        pltpu.sync_copy(ones_v, acc_sh.at[dst_v], add=True)
        return carry

    lax.fori_loop(0, N_CHUNKS, body, 0)
    plsc.subcore_barrier()
    pltpu.sync_copy(
        acc_sh.at[pl.ds(row0, ROWS_PER_TILE)],
        out_hbm.at[pl.ds(c * N_PAD + row0, ROWS_PER_TILE)],
    )


@functools.partial(
    pl.kernel,
    out_type=jax.ShapeDtypeStruct((NC * N_PAD, OUT_CH), jnp.float32),
    mesh=_mesh,
    scratch_types=[
        pltpu.VMEM((CHUNK,), jnp.int32),
        pltpu.VMEM((CHUNK,), jnp.int32),
        pltpu.VMEM((CHUNK, OUT_CH), jnp.float32),
        pltpu.VMEM_SHARED((N_PAD, OUT_CH), jnp.float32),
        pltpu.SemaphoreType.DMA,
    ],
    compiler_params=_sc_params,
)
def _hop_kernel(y_hbm, src_hbm, dst_hbm, zeros_hbm, out_hbm,
                src_v, dst_v, rows_v, acc_sh, sem):
    c = lax.axis_index("c")
    s = lax.axis_index("s")
    wid = s * NC + c
    row0 = s * ROWS_PER_TILE
    pltpu.sync_copy(zeros_hbm, acc_sh.at[pl.ds(row0, ROWS_PER_TILE)])
    plsc.subcore_barrier()
    base = wid * EPW

    def body(j, carry):
        off = base + j * CHUNK
        pltpu.sync_copy(src_hbm.at[pl.ds(off, CHUNK)], src_v)
        pltpu.sync_copy(dst_hbm.at[pl.ds(off, CHUNK)], dst_v)
        pltpu.async_copy(y_hbm.at[src_v], rows_v, sem).wait()
        pltpu.sync_copy(rows_v, acc_sh.at[dst_v], add=True)
        return carry

    lax.fori_loop(0, N_CHUNKS, body, 0)
    plsc.subcore_barrier()
    pltpu.sync_copy(
        acc_sh.at[pl.ds(row0, ROWS_PER_TILE)],
        out_hbm.at[pl.ds(c * N_PAD + row0, ROWS_PER_TILE)],
    )


_ROWS = 2000
_GRID = N // _ROWS


def _mm_body(x_ref, w_ref, o_ref):
    o_ref[...] = jnp.dot(x_ref[...], w_ref[...],
                         preferred_element_type=jnp.float32)


_matmul = pl.pallas_call(
    _mm_body,
    grid=(_GRID,),
    in_specs=[
        pl.BlockSpec((_ROWS, IN_CH), lambda i: (i, 0)),
        pl.BlockSpec((IN_CH, OUT_CH), lambda i: (0, 0)),
    ],
    out_specs=pl.BlockSpec((_ROWS, OUT_CH), lambda i: (i, 0)),
    out_shape=jax.ShapeDtypeStruct((N, OUT_CH), jnp.float32),
)


def _scale0_body(h_ref, c0_ref, c1_ref, y_ref, deg_ref):
    deg = c0_ref[...] + c1_ref[...] + 1.0
    deg_ref[...] = deg
    y_ref[...] = h_ref[...] * lax.rsqrt(deg)


_scale0 = pl.pallas_call(
    _scale0_body,
    grid=(_GRID,),
    in_specs=[
        pl.BlockSpec((_ROWS, OUT_CH), lambda i: (i, 0)),
        pl.BlockSpec((_ROWS, 1), lambda i: (i, 0)),
        pl.BlockSpec((_ROWS, 1), lambda i: (i, 0)),
    ],
    out_specs=[
        pl.BlockSpec((_ROWS, OUT_CH), lambda i: (i, 0)),
        pl.BlockSpec((_ROWS, 1), lambda i: (i, 0)),
    ],
    out_shape=[
        jax.ShapeDtypeStruct((N, OUT_CH), jnp.float32),
        jax.ShapeDtypeStruct((N, 1), jnp.float32),
    ],
)


def _mid_body(p0_ref, p1_ref, y_ref, deg_ref, o_ref):
    s = p0_ref[...] + p1_ref[...] + y_ref[...]
    o_ref[...] = s / deg_ref[...]


_mid = pl.pallas_call(
    _mid_body,
    grid=(_GRID,),
    in_specs=[
        pl.BlockSpec((_ROWS, OUT_CH), lambda i: (i, 0)),
        pl.BlockSpec((_ROWS, OUT_CH), lambda i: (i, 0)),
        pl.BlockSpec((_ROWS, OUT_CH), lambda i: (i, 0)),
        pl.BlockSpec((_ROWS, 1), lambda i: (i, 0)),
    ],
    out_specs=pl.BlockSpec((_ROWS, OUT_CH), lambda i: (i, 0)),
    out_shape=jax.ShapeDtypeStruct((N, OUT_CH), jnp.float32),
)


def _final_body(p0_ref, p1_ref, y_ref, deg_ref, b_ref, o_ref):
    s = p0_ref[...] + p1_ref[...] + y_ref[...]
    logits = s * lax.rsqrt(deg_ref[...]) + b_ref[...]
    m = jnp.max(logits, axis=1, keepdims=True)
    z = jnp.exp(logits - m)
    lse = jnp.log(jnp.sum(z, axis=1, keepdims=True)) + m
    o_ref[...] = logits - lse


_final = pl.pallas_call(
    _final_body,
    grid=(_GRID,),
    in_specs=[
        pl.BlockSpec((_ROWS, OUT_CH), lambda i: (i, 0)),
        pl.BlockSpec((_ROWS, OUT_CH), lambda i: (i, 0)),
        pl.BlockSpec((_ROWS, OUT_CH), lambda i: (i, 0)),
        pl.BlockSpec((_ROWS, 1), lambda i: (i, 0)),
        pl.BlockSpec((1, OUT_CH), lambda i: (0, 0)),
    ],
    out_specs=pl.BlockSpec((_ROWS, OUT_CH), lambda i: (i, 0)),
    out_shape=jax.ShapeDtypeStruct((N, OUT_CH), jnp.float32),
)


@jax.jit
def kernel(x, edge_index, W, b):
    src = edge_index[0].astype(jnp.int32)
    dst = edge_index[1].astype(jnp.int32)
    pad = E_PAD - E
    src_p = jnp.concatenate([src, jnp.zeros((pad,), jnp.int32)])
    dst_p = jnp.concatenate([dst, jnp.full((pad,), N, jnp.int32)])

    ones_c = jnp.ones((CHUNK,), jnp.float32)
    zeros_1 = jnp.zeros((ROWS_PER_TILE,), jnp.float32)
    zeros_2 = jnp.zeros((ROWS_PER_TILE, OUT_CH), jnp.float32)

    h0 = _matmul(x, W)

    cnt = _deg_kernel(dst_p, ones_c, zeros_1)
    cnt = cnt.reshape(NC, N_PAD)
    c0 = cnt[0, :N, None]
    c1 = cnt[1, :N, None]

    y0, deg = _scale0(h0, c0, c1)

    p1 = _hop_kernel(y0, src_p, dst_p, zeros_2).reshape(NC, N_PAD, OUT_CH)
    y1 = _mid(p1[0, :N], p1[1, :N], y0, deg)

    p2 = _hop_kernel(y1, src_p, dst_p, zeros_2).reshape(NC, N_PAD, OUT_CH)
    out = _final(p2[0, :N], p2[1, :N], y1, deg, b.reshape(1, OUT_CH))
    return out


# speedup vs baseline: 25.7063x; 1.0298x over previous
import functools

import jax
import jax.numpy as jnp
from jax import lax
from jax.experimental import pallas as pl
from jax.experimental.pallas import tpu as pltpu
from jax.experimental.pallas import tpu_sc as plsc

N = 10000
IN_CH = 128
OUT_CH = 32

NC, NS = 2, 16
NW = NC * NS
N_PAD = 10240
ROWS_PER_TILE = N_PAD // NS
CHUNK = 2048
E = 320000
EPW = 10240
E_PAD = EPW * NW
N_CHUNKS = EPW // CHUNK

_mesh = plsc.VectorSubcoreMesh(core_axis_name="c", subcore_axis_name="s")
_sc_params = pltpu.CompilerParams(use_tc_tiling_on_sc=False)


@functools.partial(
    pl.kernel,
    out_type=jax.ShapeDtypeStruct((NC * N_PAD,), jnp.float32),
    mesh=_mesh,
    scratch_types=[
        pltpu.VMEM((CHUNK,), jnp.int32),
        pltpu.VMEM((CHUNK,), jnp.float32),
        pltpu.VMEM_SHARED((N_PAD,), jnp.float32),
    ],
    compiler_params=_sc_params,
)
def _deg_kernel(dst_hbm, ones_hbm, zeros_hbm, out_hbm, dst_v, ones_v, acc_sh):
    c = lax.axis_index("c")
    s = lax.axis_index("s")
    wid = s * NC + c
    row0 = s * ROWS_PER_TILE
    pltpu.sync_copy(zeros_hbm, acc_sh.at[pl.ds(row0, ROWS_PER_TILE)])
    pltpu.sync_copy(ones_hbm, ones_v)
    plsc.subcore_barrier()
    base = wid * EPW

    def body(j, carry):
        pltpu.sync_copy(dst_hbm.at[pl.ds(base + j * CHUNK, CHUNK)], dst_v)
        pltpu.sync_copy(ones_v, acc_sh.at[dst_v], add=True)
        return carry

    lax.fori_loop(0, N_CHUNKS, body, 0)
    plsc.subcore_barrier()
    pltpu.sync_copy(
        acc_sh.at[pl.ds(row0, ROWS_PER_TILE)],
        out_hbm.at[pl.ds(c * N_PAD + row0, ROWS_PER_TILE)],
    )


@functools.partial(
    pl.kernel,
    out_type=jax.ShapeDtypeStruct((NC * N_PAD, OUT_CH), jnp.float32),
    mesh=_mesh,
    scratch_types=[
        pltpu.VMEM((CHUNK,), jnp.int32),
        pltpu.VMEM((CHUNK,), jnp.int32),
        pltpu.VMEM((CHUNK, OUT_CH), jnp.float32),
        pltpu.VMEM_SHARED((N_PAD, OUT_CH), jnp.float32),
        pltpu.SemaphoreType.DMA,
    ],
    compiler_params=_sc_params,
)
def _hop_kernel(y_hbm, src_hbm, dst_hbm, zeros_hbm, out_hbm,
                src_v, dst_v, rows_v, acc_sh, sem):
    c = lax.axis_index("c")
    s = lax.axis_index("s")
    wid = s * NC + c
    row0 = s * ROWS_PER_TILE
    pltpu.sync_copy(zeros_hbm, acc_sh.at[pl.ds(row0, ROWS_PER_TILE)])
    plsc.subcore_barrier()
    base = wid * EPW

    def body(j, carry):
        off = base + j * CHUNK
        pltpu.sync_copy(src_hbm.at[pl.ds(off, CHUNK)], src_v)
        pltpu.sync_copy(dst_hbm.at[pl.ds(off, CHUNK)], dst_v)
        pltpu.async_copy(y_hbm.at[src_v], rows_v, sem).wait()
        pltpu.sync_copy(rows_v, acc_sh.at[dst_v], add=True)
        return carry

    lax.fori_loop(0, N_CHUNKS, body, 0)
    plsc.subcore_barrier()
    pltpu.sync_copy(
        acc_sh.at[pl.ds(row0, ROWS_PER_TILE)],
        out_hbm.at[pl.ds(c * N_PAD + row0, ROWS_PER_TILE)],
    )


_ROWS = 2000
_GRID = N // _ROWS


def _mm_body(x_ref, w_ref, o_ref):
    o_ref[...] = jnp.dot(x_ref[...], w_ref[...],
                         preferred_element_type=jnp.float32)


_matmul = pl.pallas_call(
    _mm_body,
    grid=(_GRID,),
    in_specs=[
        pl.BlockSpec((_ROWS, IN_CH), lambda i: (i, 0)),
        pl.BlockSpec((IN_CH, OUT_CH), lambda i: (0, 0)),
    ],
    out_specs=pl.BlockSpec((_ROWS, OUT_CH), lambda i: (i, 0)),
    out_shape=jax.ShapeDtypeStruct((N, OUT_CH), jnp.float32),
)


def _scale0_body(h_ref, c0_ref, c1_ref, y_ref, deg_ref):
    deg = c0_ref[...] + c1_ref[...] + 1.0
    deg_ref[...] = deg
    y_ref[...] = h_ref[...] * lax.rsqrt(deg)


_scale0 = pl.pallas_call(
    _scale0_body,
    grid=(_GRID,),
    in_specs=[
        pl.BlockSpec((_ROWS, OUT_CH), lambda i: (i, 0)),
        pl.BlockSpec((_ROWS, 1), lambda i: (i, 0)),
        pl.BlockSpec((_ROWS, 1), lambda i: (i, 0)),
    ],
    out_specs=[
        pl.BlockSpec((_ROWS, OUT_CH), lambda i: (i, 0)),
        pl.BlockSpec((_ROWS, 1), lambda i: (i, 0)),
    ],
    out_shape=[
        jax.ShapeDtypeStruct((N, OUT_CH), jnp.float32),
        jax.ShapeDtypeStruct((N, 1), jnp.float32),
    ],
)


def _mid_body(p0_ref, p1_ref, y_ref, deg_ref, o_ref):
    s = p0_ref[...] + p1_ref[...] + y_ref[...]
    o_ref[...] = s / deg_ref[...]


_mid = pl.pallas_call(
    _mid_body,
    grid=(_GRID,),
    in_specs=[
        pl.BlockSpec((_ROWS, OUT_CH), lambda i: (i, 0)),
        pl.BlockSpec((_ROWS, OUT_CH), lambda i: (i, 0)),
        pl.BlockSpec((_ROWS, OUT_CH), lambda i: (i, 0)),
        pl.BlockSpec((_ROWS, 1), lambda i: (i, 0)),
    ],
    out_specs=pl.BlockSpec((_ROWS, OUT_CH), lambda i: (i, 0)),
    out_shape=jax.ShapeDtypeStruct((N, OUT_CH), jnp.float32),
)


def _final_body(p0_ref, p1_ref, y_ref, deg_ref, b_ref, o_ref):
    s = p0_ref[...] + p1_ref[...] + y_ref[...]
    logits = s * lax.rsqrt(deg_ref[...]) + b_ref[...]
    m = jnp.max(logits, axis=1, keepdims=True)
    z = jnp.exp(logits - m)
    lse = jnp.log(jnp.sum(z, axis=1, keepdims=True)) + m
    o_ref[...] = logits - lse


_final = pl.pallas_call(
    _final_body,
    grid=(_GRID,),
    in_specs=[
        pl.BlockSpec((_ROWS, OUT_CH), lambda i: (i, 0)),
        pl.BlockSpec((_ROWS, OUT_CH), lambda i: (i, 0)),
        pl.BlockSpec((_ROWS, OUT_CH), lambda i: (i, 0)),
        pl.BlockSpec((_ROWS, 1), lambda i: (i, 0)),
        pl.BlockSpec((1, OUT_CH), lambda i: (0, 0)),
    ],
    out_specs=pl.BlockSpec((_ROWS, OUT_CH), lambda i: (i, 0)),
    out_shape=jax.ShapeDtypeStruct((N, OUT_CH), jnp.float32),
)


@jax.jit
def kernel(x, edge_index, W, b):
    src = edge_index[0].astype(jnp.int32)
    dst = edge_index[1].astype(jnp.int32)
    pad = E_PAD - E
    src_p = jnp.concatenate([src, jnp.zeros((pad,), jnp.int32)])
    pad_dst = N + (jnp.arange(pad, dtype=jnp.int32) % (N_PAD - N))
    dst_p = jnp.concatenate([dst, pad_dst])

    ones_c = jnp.ones((CHUNK,), jnp.float32)
    zeros_1 = jnp.zeros((ROWS_PER_TILE,), jnp.float32)
    zeros_2 = jnp.zeros((ROWS_PER_TILE, OUT_CH), jnp.float32)

    h0 = _matmul(x, W)

    cnt = _deg_kernel(dst_p, ones_c, zeros_1)
    cnt = cnt.reshape(NC, N_PAD)
    c0 = cnt[0, :N, None]
    c1 = cnt[1, :N, None]

    y0, deg = _scale0(h0, c0, c1)

    p1 = _hop_kernel(y0, src_p, dst_p, zeros_2).reshape(NC, N_PAD, OUT_CH)
    y1 = _mid(p1[0, :N], p1[1, :N], y0, deg)

    p2 = _hop_kernel(y1, src_p, dst_p, zeros_2).reshape(NC, N_PAD, OUT_CH)
    out = _final(p2[0, :N], p2[1, :N], y1, deg, b.reshape(1, OUT_CH))
    return out


# speedup vs baseline: 43.9406x; 1.7093x over previous
import functools

import jax
import jax.numpy as jnp
from jax import lax
from jax.experimental import pallas as pl
from jax.experimental.pallas import tpu as pltpu
from jax.experimental.pallas import tpu_sc as plsc

N = 10000
IN_CH = 128
OUT_CH = 32

NC, NS = 2, 16
NW = NC * NS
N_PAD = 10240
ROWS_PER_TILE = N_PAD // NS
CHUNK = 2048
E = 320000
EPW = 10240
E_PAD = EPW * NW
N_CHUNKS = EPW // CHUNK

_mesh = plsc.VectorSubcoreMesh(core_axis_name="c", subcore_axis_name="s")
_sc_params = pltpu.CompilerParams(use_tc_tiling_on_sc=False)


@functools.partial(
    pl.kernel,
    out_type=jax.ShapeDtypeStruct((NC * N_PAD,), jnp.float32),
    mesh=_mesh,
    scratch_types=[
        pltpu.VMEM((CHUNK,), jnp.int32),
        pltpu.VMEM((CHUNK,), jnp.float32),
        pltpu.VMEM_SHARED((N_PAD,), jnp.float32),
    ],
    compiler_params=_sc_params,
)
def _deg_kernel(dst_hbm, ones_hbm, zeros_hbm, out_hbm, dst_v, ones_v, acc_sh):
    c = lax.axis_index("c")
    s = lax.axis_index("s")
    wid = s * NC + c
    row0 = s * ROWS_PER_TILE
    pltpu.sync_copy(zeros_hbm, acc_sh.at[pl.ds(row0, ROWS_PER_TILE)])
    pltpu.sync_copy(ones_hbm, ones_v)
    plsc.subcore_barrier()
    base = wid * EPW

    def body(j, carry):
        pltpu.sync_copy(dst_hbm.at[pl.ds(base + j * CHUNK, CHUNK)], dst_v)
        pltpu.sync_copy(ones_v, acc_sh.at[dst_v], add=True)
        return carry

    lax.fori_loop(0, N_CHUNKS, body, 0)
    plsc.subcore_barrier()
    pltpu.sync_copy(
        acc_sh.at[pl.ds(row0, ROWS_PER_TILE)],
        out_hbm.at[pl.ds(c * N_PAD + row0, ROWS_PER_TILE)],
    )


@functools.partial(
    pl.kernel,
    out_type=jax.ShapeDtypeStruct((NC * N_PAD, OUT_CH), jnp.float32),
    mesh=_mesh,
    scratch_types=[
        pltpu.VMEM((CHUNK,), jnp.int32),
        pltpu.VMEM((CHUNK,), jnp.int32),
        pltpu.VMEM((CHUNK, OUT_CH), jnp.float32),
        pltpu.VMEM_SHARED((N_PAD, OUT_CH), jnp.float32),
        pltpu.VMEM_SHARED((N, OUT_CH), jnp.float32),
        pltpu.SemaphoreType.DMA,
    ],
    compiler_params=_sc_params,
)
def _hop_kernel(y_hbm, src_hbm, dst_hbm, zeros_hbm, out_hbm,
                src_v, dst_v, rows_v, acc_sh, y_sh, sem):
    c = lax.axis_index("c")
    s = lax.axis_index("s")
    wid = s * NC + c
    row0 = s * ROWS_PER_TILE
    yrow0 = s * (N // NS)
    pltpu.sync_copy(y_hbm.at[pl.ds(yrow0, N // NS)],
                    y_sh.at[pl.ds(yrow0, N // NS)])
    pltpu.sync_copy(zeros_hbm, acc_sh.at[pl.ds(row0, ROWS_PER_TILE)])
    plsc.subcore_barrier()
    base = wid * EPW

    def body(j, carry):
        off = base + j * CHUNK
        pltpu.sync_copy(src_hbm.at[pl.ds(off, CHUNK)], src_v)
        pltpu.sync_copy(dst_hbm.at[pl.ds(off, CHUNK)], dst_v)
        pltpu.sync_copy(y_sh.at[src_v], rows_v)
        pltpu.sync_copy(rows_v, acc_sh.at[dst_v], add=True)
        return carry

    lax.fori_loop(0, N_CHUNKS, body, 0)
    plsc.subcore_barrier()
    pltpu.sync_copy(
        acc_sh.at[pl.ds(row0, ROWS_PER_TILE)],
        out_hbm.at[pl.ds(c * N_PAD + row0, ROWS_PER_TILE)],
    )


_ROWS = 2000
_GRID = N // _ROWS


def _mm_body(x_ref, w_ref, o_ref):
    o_ref[...] = jnp.dot(x_ref[...], w_ref[...],
                         preferred_element_type=jnp.float32)


_matmul = pl.pallas_call(
    _mm_body,
    grid=(_GRID,),
    in_specs=[
        pl.BlockSpec((_ROWS, IN_CH), lambda i: (i, 0)),
        pl.BlockSpec((IN_CH, OUT_CH), lambda i: (0, 0)),
    ],
    out_specs=pl.BlockSpec((_ROWS, OUT_CH), lambda i: (i, 0)),
    out_shape=jax.ShapeDtypeStruct((N, OUT_CH), jnp.float32),
)


def _scale0_body(h_ref, c0_ref, c1_ref, y_ref, deg_ref):
    deg = c0_ref[...] + c1_ref[...] + 1.0
    deg_ref[...] = deg
    y_ref[...] = h_ref[...] * lax.rsqrt(deg)


_scale0 = pl.pallas_call(
    _scale0_body,
    grid=(_GRID,),
    in_specs=[
        pl.BlockSpec((_ROWS, OUT_CH), lambda i: (i, 0)),
        pl.BlockSpec((_ROWS, 1), lambda i: (i, 0)),
        pl.BlockSpec((_ROWS, 1), lambda i: (i, 0)),
    ],
    out_specs=[
        pl.BlockSpec((_ROWS, OUT_CH), lambda i: (i, 0)),
        pl.BlockSpec((_ROWS, 1), lambda i: (i, 0)),
    ],
    out_shape=[
        jax.ShapeDtypeStruct((N, OUT_CH), jnp.float32),
        jax.ShapeDtypeStruct((N, 1), jnp.float32),
    ],
)


def _mid_body(p0_ref, p1_ref, y_ref, deg_ref, o_ref):
    s = p0_ref[...] + p1_ref[...] + y_ref[...]
    o_ref[...] = s / deg_ref[...]


_mid = pl.pallas_call(
    _mid_body,
    grid=(_GRID,),
    in_specs=[
        pl.BlockSpec((_ROWS, OUT_CH), lambda i: (i, 0)),
        pl.BlockSpec((_ROWS, OUT_CH), lambda i: (i, 0)),
        pl.BlockSpec((_ROWS, OUT_CH), lambda i: (i, 0)),
        pl.BlockSpec((_ROWS, 1), lambda i: (i, 0)),
    ],
    out_specs=pl.BlockSpec((_ROWS, OUT_CH), lambda i: (i, 0)),
    out_shape=jax.ShapeDtypeStruct((N, OUT_CH), jnp.float32),
)


def _final_body(p0_ref, p1_ref, y_ref, deg_ref, b_ref, o_ref):
    s = p0_ref[...] + p1_ref[...] + y_ref[...]
    logits = s * lax.rsqrt(deg_ref[...]) + b_ref[...]
    m = jnp.max(logits, axis=1, keepdims=True)
    z = jnp.exp(logits - m)
    lse = jnp.log(jnp.sum(z, axis=1, keepdims=True)) + m
    o_ref[...] = logits - lse


_final = pl.pallas_call(
    _final_body,
    grid=(_GRID,),
    in_specs=[
        pl.BlockSpec((_ROWS, OUT_CH), lambda i: (i, 0)),
        pl.BlockSpec((_ROWS, OUT_CH), lambda i: (i, 0)),
        pl.BlockSpec((_ROWS, OUT_CH), lambda i: (i, 0)),
        pl.BlockSpec((_ROWS, 1), lambda i: (i, 0)),
        pl.BlockSpec((1, OUT_CH), lambda i: (0, 0)),
    ],
    out_specs=pl.BlockSpec((_ROWS, OUT_CH), lambda i: (i, 0)),
    out_shape=jax.ShapeDtypeStruct((N, OUT_CH), jnp.float32),
)


@jax.jit
def kernel(x, edge_index, W, b):
    src = edge_index[0].astype(jnp.int32)
    dst = edge_index[1].astype(jnp.int32)
    pad = E_PAD - E
    src_p = jnp.concatenate([src, jnp.zeros((pad,), jnp.int32)])
    pad_dst = N + (jnp.arange(pad, dtype=jnp.int32) % (N_PAD - N))
    dst_p = jnp.concatenate([dst, pad_dst])

    ones_c = jnp.ones((CHUNK,), jnp.float32)
    zeros_1 = jnp.zeros((ROWS_PER_TILE,), jnp.float32)
    zeros_2 = jnp.zeros((ROWS_PER_TILE, OUT_CH), jnp.float32)

    h0 = _matmul(x, W)

    cnt = _deg_kernel(dst_p, ones_c, zeros_1)
    cnt = cnt.reshape(NC, N_PAD)
    c0 = cnt[0, :N, None]
    c1 = cnt[1, :N, None]

    y0, deg = _scale0(h0, c0, c1)

    p1 = _hop_kernel(y0, src_p, dst_p, zeros_2).reshape(NC, N_PAD, OUT_CH)
    y1 = _mid(p1[0, :N], p1[1, :N], y0, deg)

    p2 = _hop_kernel(y1, src_p, dst_p, zeros_2).reshape(NC, N_PAD, OUT_CH)
    out = _final(p2[0, :N], p2[1, :N], y1, deg, b.reshape(1, OUT_CH))
    return out


# speedup vs baseline: 46.4461x; 1.0570x over previous
import functools

import jax
import jax.numpy as jnp
from jax import lax
from jax.experimental import pallas as pl
from jax.experimental.pallas import tpu as pltpu
from jax.experimental.pallas import tpu_sc as plsc

N = 10000
IN_CH = 128
OUT_CH = 32

NC, NS = 2, 16
NW = NC * NS
N_PAD = 10240
ROWS_PER_TILE = N_PAD // NS
CHUNK = 1024
E = 320000
EPW = 10240
E_PAD = EPW * NW
N_CHUNKS = EPW // CHUNK

_mesh = plsc.VectorSubcoreMesh(core_axis_name="c", subcore_axis_name="s")
_sc_params = pltpu.CompilerParams(use_tc_tiling_on_sc=False)


@functools.partial(
    pl.kernel,
    out_type=jax.ShapeDtypeStruct((NC * N_PAD,), jnp.float32),
    mesh=_mesh,
    scratch_types=[
        pltpu.VMEM((CHUNK,), jnp.int32),
        pltpu.VMEM((CHUNK,), jnp.float32),
        pltpu.VMEM_SHARED((N_PAD,), jnp.float32),
    ],
    compiler_params=_sc_params,
)
def _deg_kernel(dst_hbm, ones_hbm, zeros_hbm, out_hbm, dst_v, ones_v, acc_sh):
    c = lax.axis_index("c")
    s = lax.axis_index("s")
    wid = s * NC + c
    row0 = s * ROWS_PER_TILE
    pltpu.sync_copy(zeros_hbm, acc_sh.at[pl.ds(row0, ROWS_PER_TILE)])
    pltpu.sync_copy(ones_hbm, ones_v)
    plsc.subcore_barrier()
    base = wid * EPW

    def body(j, carry):
        pltpu.sync_copy(dst_hbm.at[pl.ds(base + j * CHUNK, CHUNK)], dst_v)
        pltpu.sync_copy(ones_v, acc_sh.at[dst_v], add=True)
        return carry

    lax.fori_loop(0, N_CHUNKS, body, 0)
    plsc.subcore_barrier()
    pltpu.sync_copy(
        acc_sh.at[pl.ds(row0, ROWS_PER_TILE)],
        out_hbm.at[pl.ds(c * N_PAD + row0, ROWS_PER_TILE)],
    )


@functools.partial(
    pl.kernel,
    out_type=jax.ShapeDtypeStruct((NC * N_PAD, OUT_CH), jnp.float32),
    mesh=_mesh,
    scratch_types=[
        pltpu.VMEM((2, CHUNK), jnp.int32),
        pltpu.VMEM((2, CHUNK), jnp.int32),
        pltpu.VMEM((CHUNK, OUT_CH), jnp.float32),
        pltpu.VMEM((CHUNK, OUT_CH), jnp.float32),
        pltpu.VMEM_SHARED((N_PAD, OUT_CH), jnp.float32),
        pltpu.VMEM_SHARED((N, OUT_CH), jnp.float32),
        pltpu.SemaphoreType.DMA,
        pltpu.SemaphoreType.DMA,
        pltpu.SemaphoreType.DMA,
        pltpu.SemaphoreType.DMA,
        pltpu.SemaphoreType.DMA,
        pltpu.SemaphoreType.DMA,
    ],
    compiler_params=_sc_params,
)
def _hop_kernel(y_hbm, ep_hbm, zeros_hbm, out_hbm,
                idx0, idx1, rows0, rows1, acc_sh, y_sh,
                semL0, semL1, semG0, semG1, semS0, semS1):
    c = lax.axis_index("c")
    s = lax.axis_index("s")
    wid = s * NC + c
    row0 = s * ROWS_PER_TILE
    yrow0 = s * (N // NS)
    pltpu.sync_copy(y_hbm.at[pl.ds(yrow0, N // NS)],
                    y_sh.at[pl.ds(yrow0, N // NS)])
    pltpu.sync_copy(zeros_hbm, acc_sh.at[pl.ds(row0, ROWS_PER_TILE)])
    plsc.subcore_barrier()

    idx = (idx0, idx1)
    rows = (rows0, rows1)
    semL = (semL0, semL1)
    semG = (semG0, semG1)
    semS = (semS0, semS1)
    cbase = wid * N_CHUNKS

    pltpu.sync_copy(ep_hbm.at[cbase], idx0)
    pltpu.async_copy(y_sh.at[idx0.at[0]], rows0, semG0)

    def outer(jj, carry):
        for b in (0, 1):
            ob = 1 - b
            j = jj * 2 + b

            def wait_prev_scatter():
                pltpu.make_async_copy(
                    rows[ob], acc_sh.at[idx[ob].at[1]], semS[ob]).wait()

            if b == 0:
                @pl.when(jj > 0)
                def _():
                    wait_prev_scatter()
            else:
                wait_prev_scatter()

            def load_next():
                pltpu.async_copy(ep_hbm.at[cbase + j + 1], idx[ob], semL[ob])

            def start_next_gather():
                pltpu.make_async_copy(
                    ep_hbm.at[cbase + j + 1], idx[ob], semL[ob]).wait()
                pltpu.async_copy(y_sh.at[idx[ob].at[0]], rows[ob], semG[ob])

            if b == 0:
                load_next()
            else:
                @pl.when(jj < N_CHUNKS // 2 - 1)
                def _():
                    load_next()

            pltpu.make_async_copy(y_sh.at[idx[b].at[0]], rows[b],
                                  semG[b]).wait()
            pltpu.async_copy(rows[b], acc_sh.at[idx[b].at[1]], semS[b],
                             add=True)

            if b == 0:
                start_next_gather()
            else:
                @pl.when(jj < N_CHUNKS // 2 - 1)
                def _():
                    start_next_gather()
        return carry

    lax.fori_loop(0, N_CHUNKS // 2, outer, 0)
    pltpu.make_async_copy(rows[1], acc_sh.at[idx[1].at[1]], semS[1]).wait()
    plsc.subcore_barrier()
    pltpu.sync_copy(
        acc_sh.at[pl.ds(row0, ROWS_PER_TILE)],
        out_hbm.at[pl.ds(c * N_PAD + row0, ROWS_PER_TILE)],
    )


_ROWS = 2000
_GRID = N // _ROWS


def _mm_body(x_ref, w_ref, o_ref):
    o_ref[...] = jnp.dot(x_ref[...], w_ref[...],
                         preferred_element_type=jnp.float32)


_matmul = pl.pallas_call(
    _mm_body,
    grid=(_GRID,),
    in_specs=[
        pl.BlockSpec((_ROWS, IN_CH), lambda i: (i, 0)),
        pl.BlockSpec((IN_CH, OUT_CH), lambda i: (0, 0)),
    ],
    out_specs=pl.BlockSpec((_ROWS, OUT_CH), lambda i: (i, 0)),
    out_shape=jax.ShapeDtypeStruct((N, OUT_CH), jnp.float32),
)


def _scale0_body(h_ref, c0_ref, c1_ref, y_ref, deg_ref):
    deg = c0_ref[...] + c1_ref[...] + 1.0
    deg_ref[...] = deg
    y_ref[...] = h_ref[...] * lax.rsqrt(deg)


_scale0 = pl.pallas_call(
    _scale0_body,
    grid=(_GRID,),
    in_specs=[
        pl.BlockSpec((_ROWS, OUT_CH), lambda i: (i, 0)),
        pl.BlockSpec((_ROWS, 1), lambda i: (i, 0)),
        pl.BlockSpec((_ROWS, 1), lambda i: (i, 0)),
    ],
    out_specs=[
        pl.BlockSpec((_ROWS, OUT_CH), lambda i: (i, 0)),
        pl.BlockSpec((_ROWS, 1), lambda i: (i, 0)),
    ],
    out_shape=[
        jax.ShapeDtypeStruct((N, OUT_CH), jnp.float32),
        jax.ShapeDtypeStruct((N, 1), jnp.float32),
    ],
)


def _mid_body(p0_ref, p1_ref, y_ref, deg_ref, o_ref):
    s = p0_ref[...] + p1_ref[...] + y_ref[...]
    o_ref[...] = s / deg_ref[...]


_mid = pl.pallas_call(
    _mid_body,
    grid=(_GRID,),
    in_specs=[
        pl.BlockSpec((_ROWS, OUT_CH), lambda i: (i, 0)),
        pl.BlockSpec((_ROWS, OUT_CH), lambda i: (i, 0)),
        pl.BlockSpec((_ROWS, OUT_CH), lambda i: (i, 0)),
        pl.BlockSpec((_ROWS, 1), lambda i: (i, 0)),
    ],
    out_specs=pl.BlockSpec((_ROWS, OUT_CH), lambda i: (i, 0)),
    out_shape=jax.ShapeDtypeStruct((N, OUT_CH), jnp.float32),
)


def _final_body(p0_ref, p1_ref, y_ref, deg_ref, b_ref, o_ref):
    s = p0_ref[...] + p1_ref[...] + y_ref[...]
    logits = s * lax.rsqrt(deg_ref[...]) + b_ref[...]
    m = jnp.max(logits, axis=1, keepdims=True)
    z = jnp.exp(logits - m)
    lse = jnp.log(jnp.sum(z, axis=1, keepdims=True)) + m
    o_ref[...] = logits - lse


_final = pl.pallas_call(
    _final_body,
    grid=(_GRID,),
    in_specs=[
        pl.BlockSpec((_ROWS, OUT_CH), lambda i: (i, 0)),
        pl.BlockSpec((_ROWS, OUT_CH), lambda i: (i, 0)),
        pl.BlockSpec((_ROWS, OUT_CH), lambda i: (i, 0)),
        pl.BlockSpec((_ROWS, 1), lambda i: (i, 0)),
        pl.BlockSpec((1, OUT_CH), lambda i: (0, 0)),
    ],
    out_specs=pl.BlockSpec((_ROWS, OUT_CH), lambda i: (i, 0)),
    out_shape=jax.ShapeDtypeStruct((N, OUT_CH), jnp.float32),
)


@jax.jit
def kernel(x, edge_index, W, b):
    src = edge_index[0].astype(jnp.int32)
    dst = edge_index[1].astype(jnp.int32)
    pad = E_PAD - E
    src_p = jnp.concatenate([src, jnp.zeros((pad,), jnp.int32)])
    pad_dst = N + (jnp.arange(pad, dtype=jnp.int32) % (N_PAD - N))
    dst_p = jnp.concatenate([dst, pad_dst])
    ep = jnp.stack(
        [src_p.reshape(-1, CHUNK), dst_p.reshape(-1, CHUNK)], axis=1)

    ones_c = jnp.ones((CHUNK,), jnp.float32)
    zeros_1 = jnp.zeros((ROWS_PER_TILE,), jnp.float32)
    zeros_2 = jnp.zeros((ROWS_PER_TILE, OUT_CH), jnp.float32)

    h0 = _matmul(x, W)

    cnt = _deg_kernel(dst_p, ones_c, zeros_1)
    cnt = cnt.reshape(NC, N_PAD)
    c0 = cnt[0, :N, None]
    c1 = cnt[1, :N, None]

    y0, deg = _scale0(h0, c0, c1)

    p1 = _hop_kernel(y0, ep, zeros_2).reshape(NC, N_PAD, OUT_CH)
    y1 = _mid(p1[0, :N], p1[1, :N], y0, deg)

    p2 = _hop_kernel(y1, ep, zeros_2).reshape(NC, N_PAD, OUT_CH)
    out = _final(p2[0, :N], p2[1, :N], y1, deg, b.reshape(1, OUT_CH))
    return out
